# Initial kernel scaffold; baseline (speedup 1.0000x reference)
import jax
import jax.numpy as jnp
from jax.experimental import pallas as pl


def kernel(x_querry, l, x_block, e_k, e_p):
    raise NotImplementedError("write your pallas kernel here")


# speedup vs baseline: 1.3324x; 1.3324x over previous
import functools

import jax
import jax.numpy as jnp
from jax import lax
from jax.experimental import pallas as pl
from jax.experimental.pallas import tpu as pltpu
from jax.experimental.pallas import tpu_sc as plsc

B = 4096
KEY_D = 768
EMB_D = 768
POOL = 1000
PPAD = 1024
E_P_LEN = 8
HALF = (E_P_LEN // 2) * EMB_D
ROW = E_P_LEN * EMB_D

BQ = 512


def _topk_body(q_ref, ek_ref, idx_ref):
    ek = ek_ref[...]
    ekn = ek / jnp.clip(
        jnp.sqrt(jnp.sum(ek * ek, axis=1, keepdims=True)), 1e-12)
    q = q_ref[...]
    qn = q / jnp.clip(
        jnp.sqrt(jnp.sum(q * q, axis=1, keepdims=True)), 1e-12)
    s = lax.dot_general(qn, ekn, (((1,), (1,)), ((), ())),
                        preferred_element_type=jnp.float32)
    col = lax.broadcasted_iota(jnp.int32, s.shape, 1)
    s = jnp.where(col < POOL, s, -jnp.inf)
    m = jnp.max(s, axis=1, keepdims=True)
    idx = jnp.min(jnp.where(s == m, col, PPAD), axis=1)
    idx_ref[...] = idx.astype(jnp.int32).reshape(1, 1, BQ)


def _topk_call(x_querry, e_k_pad):
    nb = B // BQ
    out = pl.pallas_call(
        _topk_body,
        grid=(nb,),
        in_specs=[
            pl.BlockSpec((BQ, KEY_D), lambda i: (i, 0)),
            pl.BlockSpec((PPAD, KEY_D), lambda i: (0, 0)),
        ],
        out_specs=pl.BlockSpec((1, 1, BQ), lambda i: (i, 0, 0)),
        out_shape=jax.ShapeDtypeStruct((nb, 1, BQ), jnp.int32),
    )(x_querry, e_k_pad)
    return out.reshape(B)


def _make_gather():
    info = plsc.get_sparse_core_info()
    nc, ns = info.num_cores, info.num_subcores
    nw = nc * ns
    b_per_w = B // nw
    chunk = 16
    n_chunks = b_per_w // chunk
    mesh = plsc.VectorSubcoreMesh(core_axis_name="c", subcore_axis_name="s")

    @functools.partial(
        pl.kernel,
        mesh=mesh,
        out_type=[
            jax.ShapeDtypeStruct((B, HALF), jnp.float32),
            jax.ShapeDtypeStruct((B, HALF), jnp.float32),
        ],
        scratch_types=[
            pltpu.VMEM((b_per_w,), jnp.int32),
            pltpu.VMEM((chunk, ROW), jnp.float32),
            pltpu.SemaphoreType.DMA,
        ],
    )
    def gather(table_hbm, idx_hbm, ek_hbm, ev_hbm, idx_v, rows_v, sem):
        wid = lax.axis_index("s") * nc + lax.axis_index("c")
        base = wid * b_per_w
        pltpu.sync_copy(idx_hbm.at[pl.ds(base, b_per_w)], idx_v)
        for c in range(n_chunks):
            pltpu.async_copy(
                table_hbm.at[idx_v.at[pl.ds(c * chunk, chunk)]],
                rows_v, sem).wait()
            dst = pl.ds(base + c * chunk, chunk)
            pltpu.sync_copy(rows_v.at[:, pl.ds(0, HALF)], ek_hbm.at[dst])
            pltpu.sync_copy(rows_v.at[:, pl.ds(HALF, HALF)], ev_hbm.at[dst])

    return gather


_gather = _make_gather()


def kernel(x_querry, l, x_block, e_k, e_p):
    e_k_pad = jnp.pad(e_k, ((0, PPAD - POOL), (0, 0)))
    idx = _topk_call(x_querry, e_k_pad)
    table = e_p.reshape(POOL, ROW)
    ek_flat, ev_flat = _gather(table, idx)
    Ek = ek_flat.reshape(B, E_P_LEN // 2, EMB_D)
    Ev = ev_flat.reshape(B, E_P_LEN // 2, EMB_D)
    return (Ek, Ev, x_block)


# speedup vs baseline: 1.3336x; 1.0008x over previous
import functools

import jax
import jax.numpy as jnp
from jax import lax
from jax.experimental import pallas as pl
from jax.experimental.pallas import tpu as pltpu
from jax.experimental.pallas import tpu_sc as plsc

B = 4096
KEY_D = 768
EMB_D = 768
POOL = 1000
PPAD = 1024
E_P_LEN = 8
HALF = (E_P_LEN // 2) * EMB_D
ROW = E_P_LEN * EMB_D

BQ = 512


def _topk_body(q_ref, ek_ref, idx_ref):
    ek = ek_ref[...]
    ekn = ek / jnp.clip(
        jnp.sqrt(jnp.sum(ek * ek, axis=1, keepdims=True)), 1e-12)
    q = q_ref[...]
    qn = q / jnp.clip(
        jnp.sqrt(jnp.sum(q * q, axis=1, keepdims=True)), 1e-12)
    s = lax.dot_general(qn, ekn, (((1,), (1,)), ((), ())),
                        preferred_element_type=jnp.float32)
    col = lax.broadcasted_iota(jnp.int32, s.shape, 1)
    s = jnp.where(col < POOL, s, -jnp.inf)
    m = jnp.max(s, axis=1, keepdims=True)
    idx = jnp.min(jnp.where(s == m, col, PPAD), axis=1)
    idx_ref[...] = idx.astype(jnp.int32).reshape(1, 1, BQ)


def _topk_call(x_querry, e_k_pad):
    nb = B // BQ
    out = pl.pallas_call(
        _topk_body,
        grid=(nb,),
        in_specs=[
            pl.BlockSpec((BQ, KEY_D), lambda i: (i, 0)),
            pl.BlockSpec((PPAD, KEY_D), lambda i: (0, 0)),
        ],
        out_specs=pl.BlockSpec((1, 1, BQ), lambda i: (i, 0, 0)),
        out_shape=jax.ShapeDtypeStruct((nb, 1, BQ), jnp.int32),
    )(x_querry, e_k_pad)
    return out.reshape(B)


def _make_gather():
    info = plsc.get_sparse_core_info()
    nc, ns = info.num_cores, info.num_subcores
    nw = nc * ns
    b_per_w = B // nw
    chunk = 16
    n_chunks = b_per_w // chunk
    mesh = plsc.VectorSubcoreMesh(core_axis_name="c", subcore_axis_name="s")

    @functools.partial(
        pl.kernel,
        mesh=mesh,
        compiler_params=pltpu.CompilerParams(use_tc_tiling_on_sc=True),
        out_type=[
            jax.ShapeDtypeStruct((B, HALF), jnp.float32),
            jax.ShapeDtypeStruct((B, HALF), jnp.float32),
        ],
        scratch_types=[
            pltpu.VMEM((b_per_w,), jnp.int32),
            pltpu.VMEM((chunk, ROW), jnp.float32),
            pltpu.SemaphoreType.DMA,
        ],
    )
    def gather(table_hbm, idx_hbm, ek_hbm, ev_hbm, idx_v, rows_v, sem):
        wid = lax.axis_index("s") * nc + lax.axis_index("c")
        base = wid * b_per_w
        pltpu.sync_copy(idx_hbm.at[pl.ds(base, b_per_w)], idx_v)
        for c in range(n_chunks):
            pltpu.async_copy(
                table_hbm.at[idx_v.at[pl.ds(c * chunk, chunk)]],
                rows_v, sem).wait()
            dst = pl.ds(base + c * chunk, chunk)
            pltpu.sync_copy(rows_v.at[:, pl.ds(0, HALF)], ek_hbm.at[dst])
            pltpu.sync_copy(rows_v.at[:, pl.ds(HALF, HALF)], ev_hbm.at[dst])

    return gather


_gather = _make_gather()


def kernel(x_querry, l, x_block, e_k, e_p):
    e_k_pad = jnp.pad(e_k, ((0, PPAD - POOL), (0, 0)))
    idx = _topk_call(x_querry, e_k_pad)
    table = e_p.reshape(POOL, ROW)
    ek_flat, ev_flat = _gather(table, idx)
    Ek = ek_flat.reshape(B, E_P_LEN // 2, EMB_D)
    Ev = ev_flat.reshape(B, E_P_LEN // 2, EMB_D)
    return (Ek, Ev, x_block)


# speedup vs baseline: 2.6686x; 2.0011x over previous
import functools

import jax
import jax.numpy as jnp
from jax import lax
from jax.experimental import pallas as pl
from jax.experimental.pallas import tpu as pltpu
from jax.experimental.pallas import tpu_sc as plsc

B = 4096
KEY_D = 768
EMB_D = 768
POOL = 1000
PPAD = 1024
E_P_LEN = 8
HALF = (E_P_LEN // 2) * EMB_D
ROW = E_P_LEN * EMB_D

BQ = 512


def _topk_body(q_ref, ek_ref, idx_ref):
    ek = ek_ref[...]
    ekn = ek / jnp.clip(
        jnp.sqrt(jnp.sum(ek * ek, axis=1, keepdims=True)), 1e-12)
    q = q_ref[...]
    qn = q / jnp.clip(
        jnp.sqrt(jnp.sum(q * q, axis=1, keepdims=True)), 1e-12)
    s = lax.dot_general(qn, ekn, (((1,), (1,)), ((), ())),
                        preferred_element_type=jnp.float32)
    col = lax.broadcasted_iota(jnp.int32, s.shape, 1)
    s = jnp.where(col < POOL, s, -jnp.inf)
    m = jnp.max(s, axis=1, keepdims=True)
    idx = jnp.min(jnp.where(s == m, col, PPAD), axis=1)
    idx_ref[...] = idx.astype(jnp.int32).reshape(1, 1, BQ)


def _topk_call(x_querry, e_k_pad):
    nb = B // BQ
    out = pl.pallas_call(
        _topk_body,
        grid=(nb,),
        in_specs=[
            pl.BlockSpec((BQ, KEY_D), lambda i: (i, 0)),
            pl.BlockSpec((PPAD, KEY_D), lambda i: (0, 0)),
        ],
        out_specs=pl.BlockSpec((1, 1, BQ), lambda i: (i, 0, 0)),
        out_shape=jax.ShapeDtypeStruct((nb, 1, BQ), jnp.int32),
    )(x_querry, e_k_pad)
    return out.reshape(B)


def _make_gather():
    info = plsc.get_sparse_core_info()
    nc, ns = info.num_cores, info.num_subcores
    nw = nc * ns
    b_per_w = B // nw
    chunk = 16
    n_chunks = b_per_w // chunk
    mesh = plsc.VectorSubcoreMesh(core_axis_name="c", subcore_axis_name="s")

    half_len = E_P_LEN // 2

    @functools.partial(
        pl.kernel,
        mesh=mesh,
        compiler_params=pltpu.CompilerParams(use_tc_tiling_on_sc=True),
        out_type=[
            jax.ShapeDtypeStruct((B, half_len, EMB_D), jnp.float32),
            jax.ShapeDtypeStruct((B, half_len, EMB_D), jnp.float32),
        ],
        scratch_types=[
            pltpu.VMEM((b_per_w,), jnp.int32),
            pltpu.VMEM((chunk, E_P_LEN, EMB_D), jnp.float32),
            pltpu.SemaphoreType.DMA,
        ],
    )
    def gather(table_hbm, idx_hbm, ek_hbm, ev_hbm, idx_v, rows_v, sem):
        wid = lax.axis_index("s") * nc + lax.axis_index("c")
        base = wid * b_per_w
        pltpu.sync_copy(idx_hbm.at[pl.ds(base, b_per_w)], idx_v)
        for c in range(n_chunks):
            pltpu.async_copy(
                table_hbm.at[idx_v.at[pl.ds(c * chunk, chunk)]],
                rows_v, sem).wait()
            dst = pl.ds(base + c * chunk, chunk)
            pltpu.sync_copy(rows_v.at[:, pl.ds(0, half_len)], ek_hbm.at[dst])
            pltpu.sync_copy(rows_v.at[:, pl.ds(half_len, half_len)],
                            ev_hbm.at[dst])

    return gather


_gather = _make_gather()


def kernel(x_querry, l, x_block, e_k, e_p):
    e_k_pad = jnp.pad(e_k, ((0, PPAD - POOL), (0, 0)))
    idx = _topk_call(x_querry, e_k_pad)
    Ek, Ev = _gather(e_p, idx)
    return (Ek, Ev, x_block)


# speedup vs baseline: 2.6693x; 1.0003x over previous
import functools

import jax
import jax.numpy as jnp
from jax import lax
from jax.experimental import pallas as pl
from jax.experimental.pallas import tpu as pltpu
from jax.experimental.pallas import tpu_sc as plsc

B = 4096
KEY_D = 768
EMB_D = 768
POOL = 1000
PPAD = 1024
E_P_LEN = 8
HALF = (E_P_LEN // 2) * EMB_D
ROW = E_P_LEN * EMB_D

BQ = 512


def _topk_body(q_ref, ek_ref, idx_ref):
    ek = ek_ref[...]
    ekn = ek / jnp.clip(
        jnp.sqrt(jnp.sum(ek * ek, axis=1, keepdims=True)), 1e-12)
    q = q_ref[...]
    qn = q / jnp.clip(
        jnp.sqrt(jnp.sum(q * q, axis=1, keepdims=True)), 1e-12)
    s = lax.dot_general(qn, ekn, (((1,), (1,)), ((), ())),
                        preferred_element_type=jnp.float32)
    col = lax.broadcasted_iota(jnp.int32, s.shape, 1)
    s = jnp.where(col < POOL, s, -jnp.inf)
    m = jnp.max(s, axis=1, keepdims=True)
    idx = jnp.min(jnp.where(s == m, col, PPAD), axis=1)
    idx_ref[...] = idx.astype(jnp.int32).reshape(1, 1, BQ)


def _topk_call(x_querry, e_k_pad):
    nb = B // BQ
    out = pl.pallas_call(
        _topk_body,
        grid=(nb,),
        in_specs=[
            pl.BlockSpec((BQ, KEY_D), lambda i: (i, 0)),
            pl.BlockSpec((PPAD, KEY_D), lambda i: (0, 0)),
        ],
        out_specs=pl.BlockSpec((1, 1, BQ), lambda i: (i, 0, 0)),
        out_shape=jax.ShapeDtypeStruct((nb, 1, BQ), jnp.int32),
    )(x_querry, e_k_pad)
    return out.reshape(B)


def _make_gather():
    info = plsc.get_sparse_core_info()
    nc, ns = info.num_cores, info.num_subcores
    nw = nc * ns
    b_per_w = B // nw
    chunk = 8
    n_chunks = b_per_w // chunk
    mesh = plsc.VectorSubcoreMesh(core_axis_name="c", subcore_axis_name="s")

    half_len = E_P_LEN // 2

    @functools.partial(
        pl.kernel,
        mesh=mesh,
        compiler_params=pltpu.CompilerParams(use_tc_tiling_on_sc=True),
        out_type=[
            jax.ShapeDtypeStruct((B, half_len, EMB_D), jnp.float32),
            jax.ShapeDtypeStruct((B, half_len, EMB_D), jnp.float32),
        ],
        scratch_types=[
            pltpu.VMEM((b_per_w,), jnp.int32),
            pltpu.VMEM((2, chunk, E_P_LEN, EMB_D), jnp.float32),
            pltpu.SemaphoreType.DMA,
            pltpu.SemaphoreType.DMA,
            pltpu.SemaphoreType.DMA,
            pltpu.SemaphoreType.DMA,
        ],
    )
    def gather(table_hbm, idx_hbm, ek_hbm, ev_hbm, idx_v, rows_v,
               in_s0, in_s1, out_s0, out_s1):
        wid = lax.axis_index("s") * nc + lax.axis_index("c")
        base = wid * b_per_w
        in_sems = (in_s0, in_s1)
        out_sems = (out_s0, out_s1)
        pltpu.sync_copy(idx_hbm.at[pl.ds(base, b_per_w)], idx_v)

        def start_in(c):
            b = c & 1
            return pltpu.async_copy(
                table_hbm.at[idx_v.at[pl.ds(c * chunk, chunk)]],
                rows_v.at[b], in_sems[b])

        def start_out(c):
            b = c & 1
            dst = pl.ds(base + c * chunk, chunk)
            return (
                pltpu.async_copy(rows_v.at[b, :, pl.ds(0, half_len)],
                                 ek_hbm.at[dst], out_sems[b]),
                pltpu.async_copy(rows_v.at[b, :, pl.ds(half_len, half_len)],
                                 ev_hbm.at[dst], out_sems[b]),
            )

        pend_in = {0: start_in(0)}
        pend_out = {}
        for c in range(n_chunks):
            if c + 1 < n_chunks:
                if c - 1 in pend_out:
                    for h in pend_out.pop(c - 1):
                        h.wait()
                pend_in[c + 1] = start_in(c + 1)
            pend_in.pop(c).wait()
            pend_out[c] = start_out(c)
        for c in sorted(pend_out):
            for h in pend_out.pop(c):
                h.wait()

    return gather


_gather = _make_gather()


def kernel(x_querry, l, x_block, e_k, e_p):
    e_k_pad = jnp.pad(e_k, ((0, PPAD - POOL), (0, 0)))
    idx = _topk_call(x_querry, e_k_pad)
    Ek, Ev = _gather(e_p, idx)
    return (Ek, Ev, x_block)
